# baseline (speedup 1.0000x reference)
import functools
import math

import numpy as np

import jax
import jax.numpy as jnp
from jax.experimental import pallas as pl
from jax.experimental.pallas import tpu as pltpu

_VMEM_PHYS = 64 * 1024 * 1024


def _proj_kernel(x_ref, w_ref, b_ref, p_ref):
    x = x_ref[...].astype(jnp.bfloat16)
    y = jnp.dot(x, w_ref[...], preferred_element_type=jnp.float32)
    p_ref[...] = (y + b_ref[...]).astype(p_ref.dtype)


def _fused_projection(x2d, w12t, bias2):
    m, h = x2d.shape
    tm = m
    for cand in (1024, 512, 256, 128):
        if m % cand == 0:
            tm = cand
            break
    return pl.pallas_call(
        _proj_kernel,
        out_shape=jax.ShapeDtypeStruct((m, 2 * h), jnp.bfloat16),
        grid=(m // tm,),
        in_specs=[
            pl.BlockSpec((tm, h), lambda r: (r, 0)),
            pl.BlockSpec((h, 2 * h), lambda r: (0, 0)),
            pl.BlockSpec((1, 2 * h), lambda r: (0, 0)),
        ],
        out_specs=pl.BlockSpec((tm, 2 * h), lambda r: (r, 0)),
        compiler_params=pltpu.CompilerParams(
            dimension_semantics=("parallel",),
            vmem_limit_bytes=int(min(0.8 * _VMEM_PHYS,
                                     2 * tm * h * 4 + 2 * h * 2 * h * 2
                                     + 2 * tm * 2 * h * (2 + 4) + (4 << 20))),
        ),
        cost_estimate=pl.CostEstimate(
            flops=2 * m * h * 2 * h,
            transcendentals=0,
            bytes_accessed=m * h * 4 + 2 * h * h * 2 + 2 * m * h * 2),
    )(x2d, w12t, bias2)


def _face_kernel(h, ohi_ref, ohj_ref, p_ref, o_ref):
    batch = p_ref.shape[0]
    ohi = ohi_ref[...]
    ohj = ohj_ref[...]
    dn = (((1,), (0,)), ((), ()))
    for b in range(batch):
        a = jax.lax.dot_general(ohi, p_ref[b, :, :h], dn,
                                preferred_element_type=jnp.float32)
        c = jax.lax.dot_general(ohj, p_ref[b, :, h:], dn,
                                preferred_element_type=jnp.float32)
        o_ref[b] = jnp.tanh(a + c).astype(o_ref.dtype)


def _pick_tt(t):
    for cand in (192, 96, 64, 48, 32, 24, 16, 8):
        if t % cand == 0:
            return cand
    return t


def kernel(seq_hiddens, weight, bias):
    B, S, H = map(int, seq_hiddens.shape)
    T = S * (S + 1) // 2
    out_dtype = seq_hiddens.dtype

    w12t = jnp.concatenate([weight[:, :H].T, weight[:, H:].T],
                           axis=1).astype(jnp.bfloat16)
    bias2 = jnp.concatenate([jnp.zeros((H,), jnp.float32),
                             bias.astype(jnp.float32)]).reshape(1, 2 * H)
    x2d = seq_hiddens.reshape(B * S, H)
    p12 = _fused_projection(x2d, w12t, bias2).reshape(B, S, 2 * H)

    iu, ju = np.triu_indices(S)
    eye = np.eye(S, dtype=np.float32)
    ohi = jnp.asarray(eye[iu], dtype=jnp.bfloat16)
    ohj = jnp.asarray(eye[ju], dtype=jnp.bfloat16)

    tt = _pick_tt(T)
    nb = T // tt

    out_block = B * tt * H * 4
    vmem_limit = int(min(0.9 * _VMEM_PHYS,
                         B * S * 2 * H * 2
                         + 2 * out_block
                         + 4 * tt * S * 2
                         + 6 * tt * H * 4
                         + (6 << 20)))

    return pl.pallas_call(
        functools.partial(_face_kernel, H),
        out_shape=jax.ShapeDtypeStruct((B, T, H), out_dtype),
        grid=(nb,),
        in_specs=[
            pl.BlockSpec((tt, S), lambda t: (t, 0)),
            pl.BlockSpec((tt, S), lambda t: (t, 0)),
            pl.BlockSpec((B, S, 2 * H), lambda t: (0, 0, 0)),
        ],
        out_specs=pl.BlockSpec((B, tt, H), lambda t: (0, t, 0)),
        compiler_params=pltpu.CompilerParams(
            dimension_semantics=("parallel",),
            vmem_limit_bytes=vmem_limit),
        cost_estimate=pl.CostEstimate(
            flops=2 * 2 * B * T * S * H,
            transcendentals=B * T * H,
            bytes_accessed=B * S * 2 * H * 2 + 2 * T * S * 2 + B * T * H * 4),
    )(ohi, ohj, p12)


# speedup vs baseline: 1.0104x; 1.0104x over previous
import functools
import math

import numpy as np

import jax
import jax.numpy as jnp
from jax.experimental import pallas as pl
from jax.experimental.pallas import tpu as pltpu

_VMEM_PHYS = 64 * 1024 * 1024


def _proj_kernel(x_ref, w_ref, b_ref, p_ref):
    h = w_ref.shape[0]
    x = x_ref[...].astype(jnp.bfloat16)
    dn = (((1,), (1,)), ((), ()))
    y1 = jax.lax.dot_general(x, w_ref[:, :h], dn,
                             preferred_element_type=jnp.float32)
    y2 = jax.lax.dot_general(x, w_ref[:, h:], dn,
                             preferred_element_type=jnp.float32)
    p_ref[:, :h] = y1.astype(p_ref.dtype)
    p_ref[:, h:] = (y2 + b_ref[...]).astype(p_ref.dtype)


def _fused_projection(x2d, w12t, bias2):
    m, h = x2d.shape
    tm = m
    for cand in (1024, 512, 256, 128):
        if m % cand == 0:
            tm = cand
            break
    return pl.pallas_call(
        _proj_kernel,
        out_shape=jax.ShapeDtypeStruct((m, 2 * h), jnp.bfloat16),
        grid=(m // tm,),
        in_specs=[
            pl.BlockSpec((tm, h), lambda r: (r, 0)),
            pl.BlockSpec((h, 2 * h), lambda r: (0, 0)),
            pl.BlockSpec((1, h), lambda r: (0, 0)),
        ],
        out_specs=pl.BlockSpec((tm, 2 * h), lambda r: (r, 0)),
        compiler_params=pltpu.CompilerParams(
            dimension_semantics=("parallel",),
            vmem_limit_bytes=int(min(0.8 * _VMEM_PHYS,
                                     2 * tm * h * 4 + 2 * h * 2 * h * 2
                                     + 2 * tm * 2 * h * (2 + 4) + (4 << 20))),
        ),
        cost_estimate=pl.CostEstimate(
            flops=2 * m * h * 2 * h,
            transcendentals=0,
            bytes_accessed=m * h * 4 + 2 * h * h * 2 + 2 * m * h * 2),
    )(x2d, w12t, bias2)


def _face_kernel(h, ohi_ref, ohj_ref, p_ref, o_ref):
    batch = p_ref.shape[0]
    ohi = ohi_ref[...]
    ohj = ohj_ref[...]
    dn = (((1,), (0,)), ((), ()))
    for b in range(batch):
        a = jax.lax.dot_general(ohi, p_ref[b, :, :h], dn,
                                preferred_element_type=jnp.float32)
        c = jax.lax.dot_general(ohj, p_ref[b, :, h:], dn,
                                preferred_element_type=jnp.float32)
        o_ref[b] = jnp.tanh(a + c).astype(o_ref.dtype)


def _pick_tt(t):
    for cand in (96, 64, 48, 32, 24, 16, 8):
        if t % cand == 0:
            return cand
    return t


def kernel(seq_hiddens, weight, bias):
    B, S, H = map(int, seq_hiddens.shape)
    T = S * (S + 1) // 2
    out_dtype = seq_hiddens.dtype

    wb = weight.astype(jnp.bfloat16)
    bias1 = bias.astype(jnp.float32).reshape(1, H)
    x2d = seq_hiddens.reshape(B * S, H)
    p12 = _fused_projection(x2d, wb, bias1).reshape(B, S, 2 * H)

    iu, ju = np.triu_indices(S)
    eye = np.eye(S, dtype=np.float32)
    ohi = jnp.asarray(eye[iu], dtype=jnp.bfloat16)
    ohj = jnp.asarray(eye[ju], dtype=jnp.bfloat16)

    tt = _pick_tt(T)
    nb = T // tt

    out_block = B * tt * H * 4
    vmem_limit = int(min(0.9 * _VMEM_PHYS,
                         B * S * 2 * H * 2
                         + 2 * out_block
                         + 4 * tt * S * 2
                         + 6 * tt * H * 4
                         + (6 << 20)))

    return pl.pallas_call(
        functools.partial(_face_kernel, H),
        out_shape=jax.ShapeDtypeStruct((B, T, H), out_dtype),
        grid=(nb,),
        in_specs=[
            pl.BlockSpec((tt, S), lambda t: (t, 0)),
            pl.BlockSpec((tt, S), lambda t: (t, 0)),
            pl.BlockSpec((B, S, 2 * H), lambda t: (0, 0, 0)),
        ],
        out_specs=pl.BlockSpec((B, tt, H), lambda t: (0, t, 0)),
        compiler_params=pltpu.CompilerParams(
            dimension_semantics=("parallel",),
            vmem_limit_bytes=vmem_limit),
        cost_estimate=pl.CostEstimate(
            flops=2 * 2 * B * T * S * H,
            transcendentals=B * T * H,
            bytes_accessed=B * S * 2 * H * 2 + 2 * T * S * 2 + B * T * H * 4),
    )(ohi, ohj, p12)
